# baseline (speedup 1.0000x reference)
import functools

import jax
import jax.numpy as jnp
from jax import lax
from jax.experimental import pallas as pl
from jax.experimental.pallas import tpu as pltpu, tpu_sc as plsc

B, N, D, F, K = 64, 512, 768, 384, 40
BB = 8
KSPLIT = 256


def _dot(a, b):
    return jnp.dot(a, b, preferred_element_type=jnp.float32)


def _score_body(x_ref, w1_ref, b1_ref, w2_ref, idx_ref):
    pid = pl.program_id(0)
    x2 = x_ref[...].reshape(BB * N, D)
    xh = x2.astype(jnp.bfloat16)
    wh = w1_ref[...].astype(jnp.bfloat16)
    acc = _dot(xh[:, :KSPLIT], wh[:KSPLIT, :])
    for k0 in range(KSPLIT, D, KSPLIT):
        acc = acc + _dot(xh[:, k0 : k0 + KSPLIT], wh[k0 : k0 + KSPLIT, :])
    h = jnp.maximum(acc + b1_ref[...], 0.0)
    hb = h.astype(jnp.bfloat16)
    w2c = w2_ref[...].astype(jnp.bfloat16)

    for i in range(BB):
        s_col = _dot(hb[i * N : (i + 1) * N, :], w2c)
        s_row = s_col.T

        im = lax.broadcasted_iota(jnp.int32, (N, N), 0)
        inn = lax.broadcasted_iota(jnp.int32, (N, N), 1)
        beats = (s_col > s_row) | ((s_col == s_row) & (im < inn))
        rank = jnp.sum(beats.astype(jnp.int32), axis=0, keepdims=True)

        jk = lax.broadcasted_iota(jnp.int32, (K, N), 0)
        p = (rank == jk).astype(jnp.int32)
        nn = lax.broadcasted_iota(jnp.int32, (K, N), 1)
        idx_col = jnp.sum(p * nn, axis=1, keepdims=True)
        flat = idx_col.T + (pid * BB + i) * N
        idx_ref[i : i + 1, :] = flat


@jax.jit
def _scores_topk(patch_features, W1, b1, W2):
    grid = (B // BB,)
    return pl.pallas_call(
        _score_body,
        grid=grid,
        in_specs=[
            pl.BlockSpec((BB, N, D), lambda i: (i, 0, 0)),
            pl.BlockSpec((D, F), lambda i: (0, 0)),
            pl.BlockSpec((1, F), lambda i: (0, 0)),
            pl.BlockSpec((F, 1), lambda i: (0, 0)),
        ],
        out_specs=pl.BlockSpec((BB, K), lambda i: (i, 0)),
        out_shape=jax.ShapeDtypeStruct((B, K), jnp.int32),
    )(patch_features, W1, b1, W2)


try:
    _SC_INFO = plsc.get_sparse_core_info()
    _NC, _NS = _SC_INFO.num_cores, _SC_INFO.num_subcores
except Exception:
    _NC, _NS = 2, 16
_NW = _NC * _NS
_ROWS_PER_W = (B * K) // _NW


def _sc_gather_body(table_hbm, idx_hbm, out_hbm, idx_v, rows_v, sem):
    wid = lax.axis_index("s") * _NC + lax.axis_index("c")
    base = wid * _ROWS_PER_W
    pltpu.sync_copy(idx_hbm.at[pl.ds(base, _ROWS_PER_W)], idx_v)
    pltpu.async_copy(table_hbm.at[idx_v], rows_v, sem).wait()
    pltpu.sync_copy(rows_v, out_hbm.at[pl.ds(base, _ROWS_PER_W)])


@jax.jit
def _sc_gather(table, flat_idx):
    mesh = plsc.VectorSubcoreMesh(core_axis_name="c", subcore_axis_name="s")
    f = functools.partial(
        pl.kernel,
        mesh=mesh,
        out_type=jax.ShapeDtypeStruct((B * K, D), jnp.float32),
        scratch_types=[
            pltpu.VMEM((_ROWS_PER_W,), jnp.int32),
            pltpu.VMEM((_ROWS_PER_W, D), jnp.float32),
            pltpu.SemaphoreType.DMA,
        ],
    )(_sc_gather_body)
    return f(table, flat_idx)


def kernel(patch_features, W1, b1, W2, b2):
    del b2
    b1r = b1.reshape(1, F)
    idx = _scores_topk(patch_features, W1, b1r, W2)
    table = patch_features.reshape(B * N, D)
    rows = _sc_gather(table, idx.reshape(-1))
    return rows.reshape(B, K, D)


# speedup vs baseline: 1.0946x; 1.0946x over previous
import functools

import jax
import jax.numpy as jnp
from jax import lax
from jax.experimental import pallas as pl

B, N, D, F, K = 64, 512, 768, 384, 40
BB = 8
NH = N // 2
KSPLIT = 256


def _dot(a, b):
    return jnp.dot(a, b, preferred_element_type=jnp.float32)


def _layer1(xh, wh, b1):
    acc = _dot(xh[:, :KSPLIT], wh[:KSPLIT, :])
    for k0 in range(KSPLIT, D, KSPLIT):
        acc = acc + _dot(xh[:, k0 : k0 + KSPLIT], wh[k0 : k0 + KSPLIT, :])
    return jnp.maximum(acc + b1, 0.0)


def _body(xa_ref, xb_ref, w1_ref, b1_ref, w2_ref, out_ref):
    wh = w1_ref[...].astype(jnp.bfloat16)
    xha = xa_ref[...].reshape(BB * NH, D).astype(jnp.bfloat16)
    xhb = xb_ref[...].reshape(BB * NH, D).astype(jnp.bfloat16)
    ha = _layer1(xha, wh, b1_ref[...]).astype(jnp.bfloat16)
    hb = _layer1(xhb, wh, b1_ref[...]).astype(jnp.bfloat16)
    w2c = w2_ref[...].astype(jnp.bfloat16)

    xa3 = xha.reshape(BB, NH, D)
    xb3 = xhb.reshape(BB, NH, D)
    for i in range(BB):
        sa = _dot(ha[i * NH : (i + 1) * NH, :], w2c)
        sb = _dot(hb[i * NH : (i + 1) * NH, :], w2c)
        s_col = jnp.concatenate([sa, sb], axis=0)
        s_row = s_col.T

        im = lax.broadcasted_iota(jnp.int32, (N, N), 0)
        inn = lax.broadcasted_iota(jnp.int32, (N, N), 1)
        beats = (s_col > s_row) | ((s_col == s_row) & (im < inn))
        rank = jnp.sum(beats.astype(jnp.int32), axis=0, keepdims=True)

        jk = lax.broadcasted_iota(jnp.int32, (K, N), 0)
        p = (rank == jk).astype(jnp.bfloat16)

        out_ref[i, :, :] = _dot(p[:, :NH], xa3[i]) + _dot(p[:, NH:], xb3[i])


@jax.jit
def _run(patch_features, W1, b1, W2):
    grid = (B // BB,)
    return pl.pallas_call(
        _body,
        grid=grid,
        in_specs=[
            pl.BlockSpec((BB, NH, D), lambda i: (i, 0, 0)),
            pl.BlockSpec((BB, NH, D), lambda i: (i, 1, 0)),
            pl.BlockSpec((D, F), lambda i: (0, 0)),
            pl.BlockSpec((1, F), lambda i: (0, 0)),
            pl.BlockSpec((F, 1), lambda i: (0, 0)),
        ],
        out_specs=pl.BlockSpec((BB, K, D), lambda i: (i, 0, 0)),
        out_shape=jax.ShapeDtypeStruct((B, K, D), jnp.float32),
    )(patch_features, patch_features, W1, b1, W2)


def kernel(patch_features, W1, b1, W2, b2):
    del b2
    b1r = b1.reshape(1, F)
    return _run(patch_features, W1, b1r, W2)


# speedup vs baseline: 1.1012x; 1.0060x over previous
import functools

import jax
import jax.numpy as jnp
from jax import lax
from jax.experimental import pallas as pl

B, N, D, F, K = 64, 512, 768, 384, 40
BB = 8
NSPLIT = 4
NH = N // NSPLIT
KSPLIT = 256


def _dot(a, b):
    return jnp.dot(a, b, preferred_element_type=jnp.float32)


def _layer1(xh, wh, b1):
    acc = _dot(xh[:, :KSPLIT], wh[:KSPLIT, :])
    for k0 in range(KSPLIT, D, KSPLIT):
        acc = acc + _dot(xh[:, k0 : k0 + KSPLIT], wh[k0 : k0 + KSPLIT, :])
    return jnp.maximum(acc + b1, 0.0)


def _body(*refs):
    x_refs = refs[:NSPLIT]
    w1_ref, b1_ref, w2_ref, out_ref = refs[NSPLIT:]
    wh = w1_ref[...].astype(jnp.bfloat16)
    xhs = [r[...].reshape(BB * NH, D).astype(jnp.bfloat16) for r in x_refs]
    hs = [_layer1(xq, wh, b1_ref[...]).astype(jnp.bfloat16) for xq in xhs]
    w2c = w2_ref[...].astype(jnp.bfloat16)

    x3s = [xq.reshape(BB, NH, D) for xq in xhs]
    for i in range(BB):
        s_col = jnp.concatenate(
            [_dot(hq[i * NH : (i + 1) * NH, :], w2c) for hq in hs], axis=0
        )
        s_row = s_col.T

        im = lax.broadcasted_iota(jnp.int32, (N, N), 0)
        inn = lax.broadcasted_iota(jnp.int32, (N, N), 1)
        beats = (s_col > s_row) | ((s_col == s_row) & (im < inn))
        rank = jnp.sum(beats.astype(jnp.int32), axis=0, keepdims=True)

        jk = lax.broadcasted_iota(jnp.int32, (K, N), 0)
        p = (rank == jk).astype(jnp.bfloat16)

        acc = _dot(p[:, :NH], x3s[0][i])
        for q in range(1, NSPLIT):
            acc = acc + _dot(p[:, q * NH : (q + 1) * NH], x3s[q][i])
        out_ref[i, :, :] = acc


@jax.jit
def _run(patch_features, W1, b1, W2):
    grid = (B // BB,)
    x_specs = [
        pl.BlockSpec((BB, NH, D), functools.partial(lambda q, i: (i, q, 0), q))
        for q in range(NSPLIT)
    ]
    return pl.pallas_call(
        _body,
        grid=grid,
        in_specs=x_specs
        + [
            pl.BlockSpec((D, F), lambda i: (0, 0)),
            pl.BlockSpec((1, F), lambda i: (0, 0)),
            pl.BlockSpec((F, 1), lambda i: (0, 0)),
        ],
        out_specs=pl.BlockSpec((BB, K, D), lambda i: (i, 0, 0)),
        out_shape=jax.ShapeDtypeStruct((B, K, D), jnp.float32),
    )(*([patch_features] * NSPLIT), W1, b1, W2)


def kernel(patch_features, W1, b1, W2, b2):
    del b2
    b1r = b1.reshape(1, F)
    return _run(patch_features, W1, b1r, W2)
